# baseline (speedup 1.0000x reference)
import functools

import jax
import jax.numpy as jnp
from jax import lax
from jax.experimental import pallas as pl
from jax.experimental.pallas import tpu as pltpu

_SPLIT = 2
_BM = 200


def _gcn_body(*refs, split, bm):
    adj_refs = refs[:split]
    x_ref, w_ref, out_ref = refs[split:]
    xv = x_ref[...]
    wv = w_ref[...]
    for s in range(split):
        h = jnp.dot(adj_refs[s][...], xv, preferred_element_type=jnp.float32)
        out_ref[pl.ds(s * bm, bm), :] = lax.dot_general(
            h, wv,
            dimension_numbers=(((1,), (1,)), ((), ())),
            preferred_element_type=jnp.float32)


def kernel(adj, x, W):
    n, kdim = adj.shape
    d = x.shape[1]
    split, bm = _SPLIT, _BM
    if n % (split * bm) != 0:
        split, bm = 1, n
    adj_specs = [
        pl.BlockSpec((bm, kdim),
                     functools.partial(lambda s_, i: (split * i + s_, 0), s))
        for s in range(split)
    ]
    return pl.pallas_call(
        functools.partial(_gcn_body, split=split, bm=bm),
        grid=(n // (split * bm),),
        in_specs=adj_specs + [
            pl.BlockSpec((kdim, d), lambda i: (0, 0)),
            pl.BlockSpec((W.shape[0], W.shape[1]), lambda i: (0, 0)),
        ],
        out_specs=pl.BlockSpec((split * bm, d), lambda i: (i, 0)),
        out_shape=jax.ShapeDtypeStruct((n, d), jnp.float32),
        compiler_params=pltpu.CompilerParams(
            dimension_semantics=("arbitrary",)),
    )(*([adj] * split), x, W)


# speedup vs baseline: 1.0631x; 1.0631x over previous
import functools

import jax
import jax.numpy as jnp
from jax import lax
from jax.experimental import pallas as pl
from jax.experimental.pallas import tpu as pltpu

_BM = 400
_NBUF = 3


def _gcn_body(adj_hbm, x_ref, w_ref, out_ref, bufs, sems, *, bm, nbuf, nblk):
    i = pl.program_id(0)

    def fetch(blk, slot):
        pltpu.make_async_copy(
            adj_hbm.at[pl.ds(blk * bm, bm), :], bufs.at[slot], sems.at[slot]
        ).start()

    @pl.when(i == 0)
    def _prime():
        for s in range(nbuf - 1):
            fetch(s, s)

    slot = lax.rem(i, nbuf)
    pltpu.make_async_copy(
        adj_hbm.at[pl.ds(i * bm, bm), :], bufs.at[slot], sems.at[slot]
    ).wait()

    nxt = i + nbuf - 1

    @pl.when(nxt < nblk)
    def _refill():
        fetch(nxt, lax.rem(nxt, nbuf))

    h = jnp.dot(bufs[slot], x_ref[...], preferred_element_type=jnp.float32)
    out_ref[...] = lax.dot_general(
        h, w_ref[...],
        dimension_numbers=(((1,), (1,)), ((), ())),
        preferred_element_type=jnp.float32)


def kernel(adj, x, W):
    n, kdim = adj.shape
    d = x.shape[1]
    bm = _BM if n % _BM == 0 else n
    nblk = n // bm
    nbuf = min(_NBUF, nblk)
    return pl.pallas_call(
        functools.partial(_gcn_body, bm=bm, nbuf=nbuf, nblk=nblk),
        grid=(nblk,),
        in_specs=[
            pl.BlockSpec(memory_space=pl.ANY),
            pl.BlockSpec((kdim, d), lambda i: (0, 0)),
            pl.BlockSpec((W.shape[0], W.shape[1]), lambda i: (0, 0)),
        ],
        out_specs=pl.BlockSpec((bm, d), lambda i: (i, 0)),
        out_shape=jax.ShapeDtypeStruct((n, d), jnp.float32),
        scratch_shapes=[
            pltpu.VMEM((nbuf, bm, kdim), jnp.float32),
            pltpu.SemaphoreType.DMA((nbuf,)),
        ],
        compiler_params=pltpu.CompilerParams(
            dimension_semantics=("arbitrary",)),
    )(adj, x, W)


# speedup vs baseline: 1.0852x; 1.0207x over previous
import jax
import jax.numpy as jnp
from jax import lax
from jax.experimental import pallas as pl
from jax.experimental.pallas import tpu as pltpu


def _gcn_body(adj_ref, x_ref, w_ref, out_ref):
    h = jnp.dot(adj_ref[...], x_ref[...], preferred_element_type=jnp.float32)
    out_ref[...] = lax.dot_general(
        h, w_ref[...],
        dimension_numbers=(((1,), (1,)), ((), ())),
        preferred_element_type=jnp.float32)


def kernel(adj, x, W):
    n, kdim = adj.shape
    d = x.shape[1]
    bm = 400 if n % 400 == 0 else n
    return pl.pallas_call(
        _gcn_body,
        grid=(n // bm,),
        in_specs=[
            pl.BlockSpec((bm, kdim), lambda i: (i, 0)),
            pl.BlockSpec((kdim, d), lambda i: (0, 0)),
            pl.BlockSpec((W.shape[0], W.shape[1]), lambda i: (0, 0)),
        ],
        out_specs=pl.BlockSpec((bm, d), lambda i: (i, 0)),
        out_shape=jax.ShapeDtypeStruct((n, d), jnp.float32),
        compiler_params=pltpu.CompilerParams(
            dimension_semantics=("arbitrary",)),
    )(adj, x, W)
